# baseline (speedup 1.0000x reference)
import functools

import jax
import jax.numpy as jnp
from jax import lax
from jax.experimental import pallas as pl
from jax.experimental.pallas import tpu as pltpu
from jax.experimental.pallas import tpu_sc as plsc

DIM = 32
B = 16384
NC = 2
NS = 16
NW = NC * NS
BPW = B // NW
NCHUNK = 4
CHUNK = BPW // NCHUNK
L = 16
NG = BPW // L


def _rsqrt16(x):
    xi = plsc.bitcast(x, jnp.int32)
    one = jnp.full((L,), 1, jnp.int32)
    yi = jnp.full((L,), 0x5F3759DF, jnp.int32) - lax.shift_right_arithmetic(xi, one)
    y = plsc.bitcast(yi, jnp.float32)
    for _ in range(3):
        y = y * (1.5 - 0.5 * x * y * y)
    return jnp.minimum(y, 1e12)


def _sc_body(heads_h, tails_h, nheads_h, ntails_h, rels_h, ent_h, rel_h,
             gold_h, negd_h,
             hix, tix, aix, bix, rix,
             hrows, trows, arows, brows, rrows,
             gout, nout, sem):
    wid = lax.axis_index("s") * NC + lax.axis_index("c")

    pltpu.sync_copy(heads_h.at[wid], hix)
    pltpu.sync_copy(tails_h.at[wid], tix)
    pltpu.sync_copy(nheads_h.at[wid], aix)
    pltpu.sync_copy(ntails_h.at[wid], bix)
    pltpu.sync_copy(rels_h.at[wid], rix)

    copies = []
    for c in range(NCHUNK):
        dst = pl.ds(c * CHUNK, CHUNK)
        copies.append(pltpu.async_copy(ent_h.at[hix.at[c]], hrows.at[dst], sem))
        copies.append(pltpu.async_copy(ent_h.at[tix.at[c]], trows.at[dst], sem))
        copies.append(pltpu.async_copy(ent_h.at[aix.at[c]], arows.at[dst], sem))
        copies.append(pltpu.async_copy(ent_h.at[bix.at[c]], brows.at[dst], sem))
        copies.append(pltpu.async_copy(rel_h.at[rix.at[c]], rrows.at[dst], sem))
    for cp in copies:
        cp.wait()

    iota = lax.iota(jnp.int32, L)

    def group(g, carry):
        row = jnp.full((L,), g * L, jnp.int32) + iota
        z = jnp.zeros((L,), jnp.float32)
        hh = tt = rr = hr = ht = rt = z
        aa = bb = ar = ab = br = z
        for j in range(DIM):
            col = jnp.full((L,), j, jnp.int32)
            h = plsc.load_gather(hrows, [row, col])
            t = plsc.load_gather(trows, [row, col])
            r = plsc.load_gather(rrows, [row, col])
            a = plsc.load_gather(arows, [row, col])
            b = plsc.load_gather(brows, [row, col])
            hh += h * h
            tt += t * t
            rr += r * r
            hr += h * r
            ht += h * t
            rt += r * t
            aa += a * a
            bb += b * b
            ar += a * r
            ab += a * b
            br += b * r
        ih = _rsqrt16(hh)
        it = _rsqrt16(tt)
        g2 = hh * ih * ih + rr + tt * it * it + 2.0 * (hr * ih - ht * (ih * it) - rt * it)
        g2 = jnp.maximum(g2, 0.0)
        gval = g2 * _rsqrt16(g2)
        ia = _rsqrt16(aa)
        ib = _rsqrt16(bb)
        n2 = aa * ia * ia + rr + bb * ib * ib + 2.0 * (ar * ia - ab * (ia * ib) - br * ib)
        n2 = jnp.maximum(n2, 0.0)
        nval = n2 * _rsqrt16(n2)
        gout[pl.ds(g * L, L)] = -gval
        nout[pl.ds(g * L, L)] = -nval
        return carry

    lax.fori_loop(0, NG, group, 0)

    pltpu.sync_copy(gout, gold_h.at[wid])
    pltpu.sync_copy(nout, negd_h.at[wid])


@functools.partial(
    pl.kernel,
    mesh=plsc.VectorSubcoreMesh(core_axis_name="c", subcore_axis_name="s"),
    out_type=(
        jax.ShapeDtypeStruct((NW, BPW), jnp.float32),
        jax.ShapeDtypeStruct((NW, BPW), jnp.float32),
    ),
    scratch_types=[
        pltpu.VMEM((NCHUNK, CHUNK), jnp.int32),
        pltpu.VMEM((NCHUNK, CHUNK), jnp.int32),
        pltpu.VMEM((NCHUNK, CHUNK), jnp.int32),
        pltpu.VMEM((NCHUNK, CHUNK), jnp.int32),
        pltpu.VMEM((NCHUNK, CHUNK), jnp.int32),
        pltpu.VMEM((BPW, DIM), jnp.float32),
        pltpu.VMEM((BPW, DIM), jnp.float32),
        pltpu.VMEM((BPW, DIM), jnp.float32),
        pltpu.VMEM((BPW, DIM), jnp.float32),
        pltpu.VMEM((BPW, DIM), jnp.float32),
        pltpu.VMEM((BPW,), jnp.float32),
        pltpu.VMEM((BPW,), jnp.float32),
        pltpu.SemaphoreType.DMA,
    ],
    compiler_params=pltpu.CompilerParams(
        needs_layout_passes=False, use_tc_tiling_on_sc=False),
)
def _transe_sc(heads_h, tails_h, nheads_h, ntails_h, rels_h, ent_h, rel_h,
               gold_h, negd_h, *rest):
    _sc_body(heads_h, tails_h, nheads_h, ntails_h, rels_h, ent_h, rel_h,
             gold_h, negd_h, *rest)


def kernel(heads, tails, negative_heads, negative_tails, relations, ent_emb, rel_emb):
    def prep(ix):
        return ix.astype(jnp.int32).reshape(NW, NCHUNK, CHUNK)

    gold, negd = _transe_sc(
        prep(heads), prep(tails), prep(negative_heads), prep(negative_tails),
        prep(relations), ent_emb, rel_emb)
    return gold.reshape(B), negd.reshape(B)
